# baseline (speedup 1.0000x reference)
import functools

import jax
import jax.numpy as jnp
from jax import lax
from jax.experimental import pallas as pl
from jax.experimental.pallas import tpu as pltpu
from jax.experimental.pallas import tpu_sc as plsc

N = 10000
E = 320000
D = 128
EH = E // 2

NC = 2
NS = 16
NW = NC * NS
N_PAD = 10240
ROWS_PER_TILE = N_PAD // NS
CH_C = 80

_mesh = plsc.VectorSubcoreMesh(core_axis_name="c", subcore_axis_name="s")


def _make_gather(ne, ch):
    per_w = ne // NW
    n_it = per_w // ch
    assert per_w % ch == 0 and ch % 8 == 0 and n_it % 2 == 1

    @functools.partial(
        pl.kernel,
        out_type=jax.ShapeDtypeStruct((ne, D), jnp.float32),
        mesh=_mesh,
        scratch_types=[
            pltpu.VMEM((ch,), jnp.int32),
            pltpu.VMEM((ch,), jnp.int32),
            pltpu.VMEM((ch,), jnp.int32),
            pltpu.VMEM((ch,), jnp.int32),
            pltpu.VMEM((ch, D), jnp.float32),
            pltpu.VMEM((ch, D), jnp.float32),
            pltpu.VMEM((ch, D), jnp.float32),
            pltpu.VMEM((ch, D), jnp.float32),
            pltpu.SemaphoreType.DMA,
            pltpu.SemaphoreType.DMA,
            pltpu.SemaphoreType.DMA,
            pltpu.SemaphoreType.DMA,
        ],
    )
    def gather(a_hbm, b_hbm, dst_hbm, src_hbm, tok_hbm, g_hbm,
               idx_d0, idx_s0, idx_d1, idx_s1, bufa0, bufb0, bufa1, bufb1,
               sema0, semb0, sema1, semb1):
        del tok_hbm
        wid = lax.axis_index("c") * NS + lax.axis_index("s")
        base = wid * per_w
        slots = ((idx_d0, idx_s0, bufa0, bufb0, sema0, semb0),
                 (idx_d1, idx_s1, bufa1, bufb1, sema1, semb1))

        def issue(c, sl):
            idx_d, idx_s, bufa, bufb, sema, semb = slots[sl]
            off = base + c * ch
            pltpu.sync_copy(dst_hbm.at[pl.ds(off, ch)], idx_d)
            pltpu.sync_copy(src_hbm.at[pl.ds(off, ch)], idx_s)
            pltpu.async_copy(a_hbm.at[idx_d], bufa, sema)
            pltpu.async_copy(b_hbm.at[idx_s], bufb, semb)

        def drain(c, sl):
            idx_d, idx_s, bufa, bufb, sema, semb = slots[sl]
            off = base + c * ch
            pltpu.make_async_copy(a_hbm.at[idx_d], bufa, sema).wait()
            pltpu.make_async_copy(b_hbm.at[idx_s], bufb, semb).wait()

            def add_row(r, carry):
                for cc in range(D // 16):
                    bufa[r, pl.ds(cc * 16, 16)] = (bufa[r, pl.ds(cc * 16, 16)]
                                                   + bufb[r, pl.ds(cc * 16, 16)])
                return carry

            lax.fori_loop(0, ch, add_row, 0)
            pltpu.sync_copy(bufa, g_hbm.at[pl.ds(off, ch)])

        issue(0, 0)

        def body(it, carry):
            c0 = 2 * it
            issue(c0 + 1, 1)
            drain(c0, 0)
            issue(c0 + 2, 0)
            drain(c0 + 1, 1)
            return carry

        lax.fori_loop(0, (n_it - 1) // 2, body, 0)
        drain(n_it - 1, 0)

    return gather


def _make_scatter(ne, ch):
    per_w = ne // NW
    n_it = per_w // ch
    assert per_w % ch == 0 and ch % 8 == 0 and n_it % 2 == 1

    @functools.partial(
        pl.kernel,
        out_type=jax.ShapeDtypeStruct((NC, N_PAD, D), jnp.float32),
        mesh=_mesh,
        scratch_types=[
            pltpu.VMEM((ch,), jnp.int32),
            pltpu.VMEM((ch,), jnp.int32),
            pltpu.VMEM((ch, D), jnp.float32),
            pltpu.VMEM((ch, D), jnp.float32),
            pltpu.VMEM_SHARED((N_PAD, D), jnp.float32),
            pltpu.SemaphoreType.DMA,
            pltpu.SemaphoreType.DMA,
        ],
    )
    def scatter(m_hbm, dst_hbm, zeros_hbm, tok_hbm, s_hbm,
                idx_d0, idx_d1, bufm0, bufm1, acc, semm0, semm1):
        del tok_hbm
        cid = lax.axis_index("c")
        sid = lax.axis_index("s")
        base = (cid * NS + sid) * per_w
        row0 = sid * ROWS_PER_TILE
        pltpu.sync_copy(zeros_hbm, acc.at[pl.ds(row0, ROWS_PER_TILE)])
        plsc.subcore_barrier()
        slots = ((idx_d0, bufm0, semm0), (idx_d1, bufm1, semm1))

        def issue(c, sl):
            idx_d, bufm, semm = slots[sl]
            off = base + c * ch
            pltpu.sync_copy(dst_hbm.at[pl.ds(off, ch)], idx_d)
            pltpu.async_copy(m_hbm.at[pl.ds(off, ch)], bufm, semm)

        def process(c, sl):
            idx_d, bufm, semm = slots[sl]
            off = base + c * ch
            pltpu.make_async_copy(m_hbm.at[pl.ds(off, ch)], bufm, semm).wait()
            pltpu.sync_copy(bufm, acc.at[idx_d], add=True)

        issue(0, 0)

        def body(it, carry):
            c0 = 2 * it
            issue(c0 + 1, 1)
            process(c0, 0)
            issue(c0 + 2, 0)
            process(c0 + 1, 1)
            return carry

        lax.fori_loop(0, (n_it - 1) // 2, body, 0)
        process(n_it - 1, 0)
        plsc.subcore_barrier()
        pltpu.sync_copy(acc.at[pl.ds(row0, ROWS_PER_TILE)],
                        s_hbm.at[cid, pl.ds(row0, ROWS_PER_TILE)])

    return scatter


_sc_gather = _make_gather(EH, 40)
_sc_scatter = _make_scatter(EH, 40)


@functools.partial(
    pl.kernel,
    out_type=jax.ShapeDtypeStruct((NC, N_PAD, D), jnp.float32),
    mesh=_mesh,
    scratch_types=[
        pltpu.VMEM((CH_C,), jnp.int32),
        pltpu.VMEM((CH_C, D), jnp.float32),
        pltpu.VMEM_SHARED((N_PAD, D), jnp.float32),
    ],
)
def _sc_counts(dst_hbm, ones_hbm, zeros_hbm, c_hbm, idx_d, bufo, acc):
    cid = lax.axis_index("c")
    sid = lax.axis_index("s")
    base = (cid * NS + sid) * (E // NW)
    row0 = sid * ROWS_PER_TILE
    pltpu.sync_copy(zeros_hbm, acc.at[pl.ds(row0, ROWS_PER_TILE)])
    pltpu.sync_copy(ones_hbm, bufo)
    plsc.subcore_barrier()

    def body(j, carry):
        off = base + j * CH_C
        pltpu.sync_copy(dst_hbm.at[pl.ds(off, CH_C)], idx_d)
        pltpu.sync_copy(bufo, acc.at[idx_d], add=True)
        return carry

    lax.fori_loop(0, (E // NW) // CH_C, body, 0)
    plsc.subcore_barrier()
    pltpu.sync_copy(acc.at[pl.ds(row0, ROWS_PER_TILE)],
                    c_hbm.at[cid, pl.ds(row0, ROWS_PER_TILE)])


BN = 1000
BE = 1280

_full = lambda shape: pl.BlockSpec(shape, lambda i: (0,) * len(shape))
_rows = lambda b, w: pl.BlockSpec((b, w), lambda i: (i, 0))


def _tc_node_tables(h, wa, ba, wb):
    def body(h_ref, wa_ref, ba_ref, wb_ref, a_ref, b_ref):
        hv = h_ref[...]
        a_ref[...] = jnp.dot(hv, wa_ref[...], preferred_element_type=jnp.float32) + ba_ref[...]
        b_ref[...] = jnp.dot(hv, wb_ref[...], preferred_element_type=jnp.float32)
    return pl.pallas_call(
        body,
        grid=(N // BN,),
        in_specs=[_rows(BN, D), _full((D, D)), _full((1, D)), _full((D, D))],
        out_specs=[_rows(BN, D), _rows(BN, D)],
        out_shape=(jax.ShapeDtypeStruct((N, D), jnp.float32),
                   jax.ShapeDtypeStruct((N, D), jnp.float32)),
    )(h, wa, ba, wb)


def _tc_edge_mlp(g, w_row, v_row, w2, b2):
    def body(g_ref, w_ref, v_ref, w2_ref, b2_ref, m_ref):
        wv = lax.dot_general(w_ref[0], v_ref[...],
                             (((0,), (0,)), ((), ())),
                             preferred_element_type=jnp.float32)
        m1 = jnp.maximum(g_ref[...] + wv, 0.0).astype(jnp.bfloat16)
        m2 = jnp.dot(m1, w2_ref[...], preferred_element_type=jnp.float32) + b2_ref[...]
        m_ref[...] = jnp.maximum(m2, 0.0)
    return pl.pallas_call(
        body,
        grid=(EH // BE,),
        in_specs=[_rows(BE, D), pl.BlockSpec((1, 1, BE), lambda i: (i, 0, 0)),
                  _full((1, D)), _full((D, D)), _full((1, D))],
        out_specs=_rows(BE, D),
        out_shape=jax.ShapeDtypeStruct((EH, D), jnp.float32),
    )(g, w_row, v_row, w2, b2)


def _node_core(h_ref, sparts, c0, c1, wn1a, wn1b, bn1, wn2, bn2):
    cnt = jnp.maximum(c0[...][:, :1] + c1[...][:, :1], 1.0)
    ssum = sparts[0][...] + sparts[1][...] + sparts[2][...] + sparts[3][...]
    aggr = ssum * (1.0 / cnt)
    hv = h_ref[...]
    u = jnp.dot(hv, wn1a[...], preferred_element_type=jnp.float32)
    u = u + jnp.dot(aggr, wn1b[...], preferred_element_type=jnp.float32) + bn1[...]
    u = jnp.maximum(u, 0.0)
    hn = jnp.dot(u, wn2[...], preferred_element_type=jnp.float32) + bn2[...]
    return jnp.maximum(hn, 0.0)


def _tc_node_update(h, s00, s01, s10, s11, c0, c1,
                    wn1a, wn1b, bn1, wn2, bn2, wa, ba, wb):
    def body(h_ref, s00, s01, s10, s11, c0, c1, wn1a, wn1b, bn1, wn2, bn2,
             wa_ref, ba_ref, wb_ref, h_out, a_out, b_out):
        hn = _node_core(h_ref, (s00, s01, s10, s11), c0, c1,
                        wn1a, wn1b, bn1, wn2, bn2)
        h_out[...] = hn
        a_out[...] = jnp.dot(hn, wa_ref[...], preferred_element_type=jnp.float32) + ba_ref[...]
        b_out[...] = jnp.dot(hn, wb_ref[...], preferred_element_type=jnp.float32)
    return pl.pallas_call(
        body,
        grid=(N // BN,),
        in_specs=[_rows(BN, D), _rows(BN, D), _rows(BN, D), _rows(BN, D),
                  _rows(BN, D), _rows(BN, D), _rows(BN, D),
                  _full((D, D)), _full((D, D)), _full((1, D)),
                  _full((D, D)), _full((1, D)),
                  _full((D, D)), _full((1, D)), _full((D, D))],
        out_specs=[_rows(BN, D), _rows(BN, D), _rows(BN, D)],
        out_shape=(jax.ShapeDtypeStruct((N, D), jnp.float32),
                   jax.ShapeDtypeStruct((N, D), jnp.float32),
                   jax.ShapeDtypeStruct((N, D), jnp.float32)),
    )(h, s00, s01, s10, s11, c0, c1, wn1a, wn1b, bn1, wn2, bn2, wa, ba, wb)


def _tc_node_final(h, s00, s01, s10, s11, c0, c1,
                   wn1a, wn1b, bn1, wn2, bn2, g, bta):
    def body(h_ref, s00, s01, s10, s11, c0, c1, wn1a, wn1b, bn1, wn2, bn2,
             g_ref, bta_ref, y_out):
        hn = _node_core(h_ref, (s00, s01, s10, s11), c0, c1,
                        wn1a, wn1b, bn1, wn2, bn2)
        mu = jnp.mean(hn, axis=1, keepdims=True)
        dlt = hn - mu
        var = jnp.mean(dlt * dlt, axis=1, keepdims=True)
        y_out[...] = dlt * lax.rsqrt(var + 1e-5) * g_ref[...] + bta_ref[...]
    return pl.pallas_call(
        body,
        grid=(N // BN,),
        in_specs=[_rows(BN, D), _rows(BN, D), _rows(BN, D), _rows(BN, D),
                  _rows(BN, D), _rows(BN, D), _rows(BN, D),
                  _full((D, D)), _full((D, D)), _full((1, D)),
                  _full((D, D)), _full((1, D)),
                  _full((1, D)), _full((1, D))],
        out_specs=_rows(BN, D),
        out_shape=jax.ShapeDtypeStruct((N, D), jnp.float32),
    )(h, s00, s01, s10, s11, c0, c1, wn1a, wn1b, bn1, wn2, bn2, g, bta)


def kernel(x, edge_index, edge_weight, params):
    src = edge_index[0].astype(jnp.int32)
    dst = edge_index[1].astype(jnp.int32)
    src_h = (src[:EH], src[EH:])
    dst_h = (dst[:EH], dst[EH:])
    w_h = (edge_weight[:EH].reshape(EH // BE, 1, BE),
           edge_weight[EH:].reshape(EH // BE, 1, BE))

    zeros_d = jnp.zeros((ROWS_PER_TILE, D), jnp.float32)
    ones_d = jnp.ones((CH_C, D), jnp.float32)

    cpart = _sc_counts(dst, ones_d, zeros_d)
    c0, c1 = cpart[0, :N], cpart[1, :N]
    tok = cpart[0, :8, 0]

    layers = params["layers"]

    h = x
    we1 = layers[0]["We1"]
    a, b = _tc_node_tables(h, we1[:D], layers[0]["be1"].reshape(1, D), we1[D:2 * D])
    for i, p in enumerate(layers):
        v_row = p["We1"][2 * D:2 * D + 1]
        w2_bf = p["We2"].astype(jnp.bfloat16)
        b2 = p["be2"].reshape(1, D)

        g0 = _sc_gather(a, b, dst_h[0], src_h[0], tok)
        g1 = _sc_gather(a, b, dst_h[1], src_h[1], g0[:8, 0])
        m0 = _tc_edge_mlp(g0, w_h[0], v_row, w2_bf, b2)
        m1 = _tc_edge_mlp(g1, w_h[1], v_row, w2_bf, b2)
        sp0 = _sc_scatter(m0, dst_h[0], zeros_d, g1[:8, 0])
        sp1 = _sc_scatter(m1, dst_h[1], zeros_d, sp0[0, :8, 0])
        tok = sp1[0, :8, 0]

        wn1 = p["Wn1"]
        args = (h, sp0[0, :N], sp0[1, :N], sp1[0, :N], sp1[1, :N], c0, c1,
                wn1[:D], wn1[D:], p["bn1"].reshape(1, D),
                p["Wn2"], p["bn2"].reshape(1, D))
        if i + 1 < len(layers):
            nxt = layers[i + 1]
            we1n = nxt["We1"]
            h, a, b = _tc_node_update(*args, we1n[:D],
                                      nxt["be1"].reshape(1, D), we1n[D:2 * D])
        else:
            h = _tc_node_final(*args, params["ln_scale"].reshape(1, D),
                               params["ln_bias"].reshape(1, D))
    return h


# speedup vs baseline: 1.2371x; 1.2371x over previous
import functools

import jax
import jax.numpy as jnp
from jax import lax
from jax.experimental import pallas as pl
from jax.experimental.pallas import tpu as pltpu
from jax.experimental.pallas import tpu_sc as plsc

N = 10000
E = 320000
D = 128
EH = E // 2

NC = 2
NS = 16
NW = NC * NS
N_PAD = 10240
ROWS_PER_TILE = N_PAD // NS
CH_C = 80

_mesh = plsc.VectorSubcoreMesh(core_axis_name="c", subcore_axis_name="s")


def _make_gather(ne, ch):
    per_w = ne // NW
    n_it = per_w // ch
    assert per_w % ch == 0 and ch % 8 == 0 and n_it % 2 == 1

    @functools.partial(
        pl.kernel,
        out_type=jax.ShapeDtypeStruct((ne, D), jnp.float32),
        mesh=_mesh,
        scratch_types=[
            pltpu.VMEM((n_it, ch), jnp.int32),
            pltpu.VMEM((n_it, ch), jnp.int32),
            pltpu.VMEM((ch, D), jnp.float32),
            pltpu.VMEM((ch, D), jnp.float32),
            pltpu.VMEM((ch, D), jnp.float32),
            pltpu.VMEM((ch, D), jnp.float32),
            pltpu.SemaphoreType.DMA,
            pltpu.SemaphoreType.DMA,
            pltpu.SemaphoreType.DMA,
            pltpu.SemaphoreType.DMA,
        ],
    )
    def gather(a_hbm, b_hbm, dst2_hbm, src2_hbm, tok_hbm, g_hbm,
               idx_d2, idx_s2, bufa0, bufb0, bufa1, bufb1,
               sema0, semb0, sema1, semb1):
        del tok_hbm
        wid = lax.axis_index("c") * NS + lax.axis_index("s")
        base = wid * per_w
        pltpu.sync_copy(dst2_hbm.at[wid], idx_d2)
        pltpu.sync_copy(src2_hbm.at[wid], idx_s2)
        slots = ((bufa0, bufb0, sema0, semb0),
                 (bufa1, bufb1, sema1, semb1))

        def issue(c, sl):
            bufa, bufb, sema, semb = slots[sl]
            pltpu.async_copy(a_hbm.at[idx_d2.at[c]], bufa, sema)
            pltpu.async_copy(b_hbm.at[idx_s2.at[c]], bufb, semb)

        def drain(c, sl):
            bufa, bufb, sema, semb = slots[sl]
            pltpu.make_async_copy(a_hbm.at[idx_d2.at[c]], bufa, sema).wait()
            pltpu.make_async_copy(b_hbm.at[idx_s2.at[c]], bufb, semb).wait()

            def add_row(r, carry):
                for cc in range(D // 16):
                    bufa[r, pl.ds(cc * 16, 16)] = (bufa[r, pl.ds(cc * 16, 16)]
                                                   + bufb[r, pl.ds(cc * 16, 16)])
                return carry

            lax.fori_loop(0, ch, add_row, 0)
            pltpu.sync_copy(bufa, g_hbm.at[pl.ds(base + c * ch, ch)])

        issue(0, 0)

        def body(it, carry):
            c0 = 2 * it
            issue(c0 + 1, 1)
            drain(c0, 0)
            issue(c0 + 2, 0)
            drain(c0 + 1, 1)
            return carry

        lax.fori_loop(0, (n_it - 1) // 2, body, 0)
        drain(n_it - 1, 0)

    return gather


def _make_scatter(ne, ch):
    per_w = ne // NW
    n_it = per_w // ch
    assert per_w % ch == 0 and ch % 8 == 0 and n_it % 2 == 1

    @functools.partial(
        pl.kernel,
        out_type=jax.ShapeDtypeStruct((NC, N_PAD, D), jnp.float32),
        mesh=_mesh,
        scratch_types=[
            pltpu.VMEM((n_it, ch), jnp.int32),
            pltpu.VMEM((ch, D), jnp.float32),
            pltpu.VMEM((ch, D), jnp.float32),
            pltpu.VMEM_SHARED((N_PAD, D), jnp.float32),
            pltpu.SemaphoreType.DMA,
            pltpu.SemaphoreType.DMA,
        ],
    )
    def scatter(m_hbm, dst2_hbm, zeros_hbm, tok_hbm, s_hbm,
                idx_d2, bufm0, bufm1, acc, semm0, semm1):
        del tok_hbm
        cid = lax.axis_index("c")
        sid = lax.axis_index("s")
        wid = cid * NS + sid
        base = wid * per_w
        row0 = sid * ROWS_PER_TILE
        pltpu.sync_copy(dst2_hbm.at[wid], idx_d2)
        pltpu.sync_copy(zeros_hbm, acc.at[pl.ds(row0, ROWS_PER_TILE)])
        plsc.subcore_barrier()
        slots = ((bufm0, semm0), (bufm1, semm1))

        def issue(c, sl):
            bufm, semm = slots[sl]
            pltpu.async_copy(m_hbm.at[pl.ds(base + c * ch, ch)], bufm, semm)

        def process(c, sl):
            bufm, semm = slots[sl]
            pltpu.make_async_copy(m_hbm.at[pl.ds(base + c * ch, ch)],
                                  bufm, semm).wait()
            pltpu.sync_copy(bufm, acc.at[idx_d2.at[c]], add=True)

        issue(0, 0)

        def body(it, carry):
            c0 = 2 * it
            issue(c0 + 1, 1)
            process(c0, 0)
            issue(c0 + 2, 0)
            process(c0 + 1, 1)
            return carry

        lax.fori_loop(0, (n_it - 1) // 2, body, 0)
        process(n_it - 1, 0)
        plsc.subcore_barrier()
        pltpu.sync_copy(acc.at[pl.ds(row0, ROWS_PER_TILE)],
                        s_hbm.at[cid, pl.ds(row0, ROWS_PER_TILE)])

    return scatter


_sc_gather = _make_gather(EH, 40)
_sc_scatter = _make_scatter(EH, 40)


@functools.partial(
    pl.kernel,
    out_type=jax.ShapeDtypeStruct((NC, N_PAD, D), jnp.float32),
    mesh=_mesh,
    scratch_types=[
        pltpu.VMEM((CH_C,), jnp.int32),
        pltpu.VMEM((CH_C, D), jnp.float32),
        pltpu.VMEM_SHARED((N_PAD, D), jnp.float32),
    ],
)
def _sc_counts(dst_hbm, ones_hbm, zeros_hbm, c_hbm, idx_d, bufo, acc):
    cid = lax.axis_index("c")
    sid = lax.axis_index("s")
    base = (cid * NS + sid) * (E // NW)
    row0 = sid * ROWS_PER_TILE
    pltpu.sync_copy(zeros_hbm, acc.at[pl.ds(row0, ROWS_PER_TILE)])
    pltpu.sync_copy(ones_hbm, bufo)
    plsc.subcore_barrier()

    def body(j, carry):
        off = base + j * CH_C
        pltpu.sync_copy(dst_hbm.at[pl.ds(off, CH_C)], idx_d)
        pltpu.sync_copy(bufo, acc.at[idx_d], add=True)
        return carry

    lax.fori_loop(0, (E // NW) // CH_C, body, 0)
    plsc.subcore_barrier()
    pltpu.sync_copy(acc.at[pl.ds(row0, ROWS_PER_TILE)],
                    c_hbm.at[cid, pl.ds(row0, ROWS_PER_TILE)])


BN = 1000
BE = 1280

_full = lambda shape: pl.BlockSpec(shape, lambda i: (0,) * len(shape))
_rows = lambda b, w: pl.BlockSpec((b, w), lambda i: (i, 0))


def _tc_node_tables(h, wa, ba, wb):
    def body(h_ref, wa_ref, ba_ref, wb_ref, a_ref, b_ref):
        hv = h_ref[...]
        a_ref[...] = jnp.dot(hv, wa_ref[...], preferred_element_type=jnp.float32) + ba_ref[...]
        b_ref[...] = jnp.dot(hv, wb_ref[...], preferred_element_type=jnp.float32)
    return pl.pallas_call(
        body,
        grid=(N // BN,),
        in_specs=[_rows(BN, D), _full((D, D)), _full((1, D)), _full((D, D))],
        out_specs=[_rows(BN, D), _rows(BN, D)],
        out_shape=(jax.ShapeDtypeStruct((N, D), jnp.float32),
                   jax.ShapeDtypeStruct((N, D), jnp.float32)),
    )(h, wa, ba, wb)


def _tc_edge_mlp(g, w_row, v_row, w2, b2):
    def body(g_ref, w_ref, v_ref, w2_ref, b2_ref, m_ref):
        wv = lax.dot_general(w_ref[0], v_ref[...],
                             (((0,), (0,)), ((), ())),
                             preferred_element_type=jnp.float32)
        m1 = jnp.maximum(g_ref[...] + wv, 0.0).astype(jnp.bfloat16)
        m2 = jnp.dot(m1, w2_ref[...], preferred_element_type=jnp.float32) + b2_ref[...]
        m_ref[...] = jnp.maximum(m2, 0.0)
    return pl.pallas_call(
        body,
        grid=(EH // BE,),
        in_specs=[_rows(BE, D), pl.BlockSpec((1, 1, BE), lambda i: (i, 0, 0)),
                  _full((1, D)), _full((D, D)), _full((1, D))],
        out_specs=_rows(BE, D),
        out_shape=jax.ShapeDtypeStruct((EH, D), jnp.float32),
    )(g, w_row, v_row, w2, b2)


def _node_core(h_ref, sparts, c0, c1, wn1a, wn1b, bn1, wn2, bn2):
    cnt = jnp.maximum(c0[...][:, :1] + c1[...][:, :1], 1.0)
    ssum = sparts[0][...] + sparts[1][...] + sparts[2][...] + sparts[3][...]
    aggr = ssum * (1.0 / cnt)
    hv = h_ref[...]
    u = jnp.dot(hv, wn1a[...], preferred_element_type=jnp.float32)
    u = u + jnp.dot(aggr, wn1b[...], preferred_element_type=jnp.float32) + bn1[...]
    u = jnp.maximum(u, 0.0)
    hn = jnp.dot(u, wn2[...], preferred_element_type=jnp.float32) + bn2[...]
    return jnp.maximum(hn, 0.0)


def _tc_node_update(h, s00, s01, s10, s11, c0, c1,
                    wn1a, wn1b, bn1, wn2, bn2, wa, ba, wb):
    def body(h_ref, s00, s01, s10, s11, c0, c1, wn1a, wn1b, bn1, wn2, bn2,
             wa_ref, ba_ref, wb_ref, h_out, a_out, b_out):
        hn = _node_core(h_ref, (s00, s01, s10, s11), c0, c1,
                        wn1a, wn1b, bn1, wn2, bn2)
        h_out[...] = hn
        a_out[...] = jnp.dot(hn, wa_ref[...], preferred_element_type=jnp.float32) + ba_ref[...]
        b_out[...] = jnp.dot(hn, wb_ref[...], preferred_element_type=jnp.float32)
    return pl.pallas_call(
        body,
        grid=(N // BN,),
        in_specs=[_rows(BN, D), _rows(BN, D), _rows(BN, D), _rows(BN, D),
                  _rows(BN, D), _rows(BN, D), _rows(BN, D),
                  _full((D, D)), _full((D, D)), _full((1, D)),
                  _full((D, D)), _full((1, D)),
                  _full((D, D)), _full((1, D)), _full((D, D))],
        out_specs=[_rows(BN, D), _rows(BN, D), _rows(BN, D)],
        out_shape=(jax.ShapeDtypeStruct((N, D), jnp.float32),
                   jax.ShapeDtypeStruct((N, D), jnp.float32),
                   jax.ShapeDtypeStruct((N, D), jnp.float32)),
    )(h, s00, s01, s10, s11, c0, c1, wn1a, wn1b, bn1, wn2, bn2, wa, ba, wb)


def _tc_node_final(h, s00, s01, s10, s11, c0, c1,
                   wn1a, wn1b, bn1, wn2, bn2, g, bta):
    def body(h_ref, s00, s01, s10, s11, c0, c1, wn1a, wn1b, bn1, wn2, bn2,
             g_ref, bta_ref, y_out):
        hn = _node_core(h_ref, (s00, s01, s10, s11), c0, c1,
                        wn1a, wn1b, bn1, wn2, bn2)
        mu = jnp.mean(hn, axis=1, keepdims=True)
        dlt = hn - mu
        var = jnp.mean(dlt * dlt, axis=1, keepdims=True)
        y_out[...] = dlt * lax.rsqrt(var + 1e-5) * g_ref[...] + bta_ref[...]
    return pl.pallas_call(
        body,
        grid=(N // BN,),
        in_specs=[_rows(BN, D), _rows(BN, D), _rows(BN, D), _rows(BN, D),
                  _rows(BN, D), _rows(BN, D), _rows(BN, D),
                  _full((D, D)), _full((D, D)), _full((1, D)),
                  _full((D, D)), _full((1, D)),
                  _full((1, D)), _full((1, D))],
        out_specs=_rows(BN, D),
        out_shape=jax.ShapeDtypeStruct((N, D), jnp.float32),
    )(h, s00, s01, s10, s11, c0, c1, wn1a, wn1b, bn1, wn2, bn2, g, bta)


def kernel(x, edge_index, edge_weight, params):
    src = edge_index[0].astype(jnp.int32)
    dst = edge_index[1].astype(jnp.int32)
    src_h = (src[:EH].reshape(NW, EH // (NW * 40), 40),
             src[EH:].reshape(NW, EH // (NW * 40), 40))
    dst_h = (dst[:EH].reshape(NW, EH // (NW * 40), 40),
             dst[EH:].reshape(NW, EH // (NW * 40), 40))
    w_h = (edge_weight[:EH].reshape(EH // BE, 1, BE),
           edge_weight[EH:].reshape(EH // BE, 1, BE))

    zeros_d = jnp.zeros((ROWS_PER_TILE, D), jnp.float32)
    ones_d = jnp.ones((CH_C, D), jnp.float32)

    cpart = _sc_counts(dst, ones_d, zeros_d)
    c0, c1 = cpart[0, :N], cpart[1, :N]
    tok = cpart[0, :8, 0]

    layers = params["layers"]

    h = x
    we1 = layers[0]["We1"]
    a, b = _tc_node_tables(h, we1[:D], layers[0]["be1"].reshape(1, D), we1[D:2 * D])
    for i, p in enumerate(layers):
        v_row = p["We1"][2 * D:2 * D + 1]
        w2_bf = p["We2"].astype(jnp.bfloat16)
        b2 = p["be2"].reshape(1, D)

        g0 = _sc_gather(a, b, dst_h[0], src_h[0], tok)
        g1 = _sc_gather(a, b, dst_h[1], src_h[1], g0[:8, 0])
        m0 = _tc_edge_mlp(g0, w_h[0], v_row, w2_bf, b2)
        m1 = _tc_edge_mlp(g1, w_h[1], v_row, w2_bf, b2)
        sp0 = _sc_scatter(m0, dst_h[0], zeros_d, g1[:8, 0])
        sp1 = _sc_scatter(m1, dst_h[1], zeros_d, sp0[0, :8, 0])
        tok = sp1[0, :8, 0]

        wn1 = p["Wn1"]
        args = (h, sp0[0, :N], sp0[1, :N], sp1[0, :N], sp1[1, :N], c0, c1,
                wn1[:D], wn1[D:], p["bn1"].reshape(1, D),
                p["Wn2"], p["bn2"].reshape(1, D))
        if i + 1 < len(layers):
            nxt = layers[i + 1]
            we1n = nxt["We1"]
            h, a, b = _tc_node_update(*args, we1n[:D],
                                      nxt["be1"].reshape(1, D), we1n[D:2 * D])
        else:
            h = _tc_node_final(*args, params["ln_scale"].reshape(1, D),
                               params["ln_bias"].reshape(1, D))
    return h


# speedup vs baseline: 1.2745x; 1.0302x over previous
import functools

import jax
import jax.numpy as jnp
from jax import lax
from jax.experimental import pallas as pl
from jax.experimental.pallas import tpu as pltpu
from jax.experimental.pallas import tpu_sc as plsc

N = 10000
E = 320000
D = 128
EH = E // 2

NC = 2
NS = 16
NW = NC * NS
N_PAD = 10240
ROWS_PER_TILE = N_PAD // NS
CH_C = 80

_mesh = plsc.VectorSubcoreMesh(core_axis_name="c", subcore_axis_name="s")


def _make_gather(ne, ch):
    per_w = ne // NW
    n_it = per_w // ch
    assert per_w % ch == 0 and ch % 8 == 0 and n_it % 2 == 1

    @functools.partial(
        pl.kernel,
        out_type=jax.ShapeDtypeStruct((ne, D), jnp.float32),
        mesh=_mesh,
        scratch_types=[
            pltpu.VMEM((n_it, ch), jnp.int32),
            pltpu.VMEM((n_it, ch), jnp.int32),
            pltpu.VMEM((ch, D), jnp.float32),
            pltpu.VMEM((ch, D), jnp.float32),
            pltpu.VMEM((ch, D), jnp.float32),
            pltpu.VMEM((ch, D), jnp.float32),
            pltpu.SemaphoreType.DMA,
            pltpu.SemaphoreType.DMA,
            pltpu.SemaphoreType.DMA,
            pltpu.SemaphoreType.DMA,
        ],
    )
    def gather(a_hbm, b_hbm, dst2_hbm, src2_hbm, tok_hbm, g_hbm,
               idx_d2, idx_s2, bufa0, bufb0, bufa1, bufb1,
               sema0, semb0, sema1, semb1):
        del tok_hbm
        wid = lax.axis_index("c") * NS + lax.axis_index("s")
        base = wid * per_w
        pltpu.sync_copy(dst2_hbm.at[wid], idx_d2)
        pltpu.sync_copy(src2_hbm.at[wid], idx_s2)
        slots = ((bufa0, bufb0, sema0, semb0),
                 (bufa1, bufb1, sema1, semb1))

        def issue(c, sl):
            bufa, bufb, sema, semb = slots[sl]
            pltpu.async_copy(a_hbm.at[idx_d2.at[c]], bufa, sema)
            pltpu.async_copy(b_hbm.at[idx_s2.at[c]], bufb, semb)

        def drain(c, sl):
            bufa, bufb, sema, semb = slots[sl]
            pltpu.make_async_copy(a_hbm.at[idx_d2.at[c]], bufa, sema).wait()
            pltpu.make_async_copy(b_hbm.at[idx_s2.at[c]], bufb, semb).wait()

            def add_row(r, carry):
                for cc in range(D // 16):
                    bufa[r, pl.ds(cc * 16, 16)] = (bufa[r, pl.ds(cc * 16, 16)]
                                                   + bufb[r, pl.ds(cc * 16, 16)])
                return carry

            lax.fori_loop(0, ch, add_row, 0)
            pltpu.sync_copy(bufa, g_hbm.at[pl.ds(base + c * ch, ch)])

        issue(0, 0)

        def body(it, carry):
            c0 = 2 * it
            issue(c0 + 1, 1)
            drain(c0, 0)
            issue(c0 + 2, 0)
            drain(c0 + 1, 1)
            return carry

        lax.fori_loop(0, (n_it - 1) // 2, body, 0)
        drain(n_it - 1, 0)

    return gather


def _make_scatter(ne, ch):
    per_w = ne // NW
    n_it = per_w // ch
    assert per_w % ch == 0 and ch % 8 == 0 and n_it % 2 == 1

    @functools.partial(
        pl.kernel,
        out_type=jax.ShapeDtypeStruct((NC, N_PAD, D), jnp.float32),
        mesh=_mesh,
        scratch_types=[
            pltpu.VMEM((n_it, ch), jnp.int32),
            pltpu.VMEM((ch, D), jnp.float32),
            pltpu.VMEM((ch, D), jnp.float32),
            pltpu.VMEM_SHARED((N_PAD, D), jnp.float32),
            pltpu.SemaphoreType.DMA,
            pltpu.SemaphoreType.DMA,
        ],
    )
    def scatter(m_hbm, dst2_hbm, zeros_hbm, tok_hbm, s_hbm,
                idx_d2, bufm0, bufm1, acc, semm0, semm1):
        del tok_hbm
        cid = lax.axis_index("c")
        sid = lax.axis_index("s")
        wid = cid * NS + sid
        base = wid * per_w
        row0 = sid * ROWS_PER_TILE
        pltpu.sync_copy(dst2_hbm.at[wid], idx_d2)
        pltpu.sync_copy(zeros_hbm, acc.at[pl.ds(row0, ROWS_PER_TILE)])
        plsc.subcore_barrier()
        slots = ((bufm0, semm0), (bufm1, semm1))

        def issue(c, sl):
            bufm, semm = slots[sl]
            pltpu.async_copy(m_hbm.at[pl.ds(base + c * ch, ch)], bufm, semm)

        def process(c, sl):
            bufm, semm = slots[sl]
            pltpu.make_async_copy(m_hbm.at[pl.ds(base + c * ch, ch)],
                                  bufm, semm).wait()
            pltpu.sync_copy(bufm, acc.at[idx_d2.at[c]], add=True)

        issue(0, 0)

        def body(it, carry):
            c0 = 2 * it
            issue(c0 + 1, 1)
            process(c0, 0)
            issue(c0 + 2, 0)
            process(c0 + 1, 1)
            return carry

        lax.fori_loop(0, (n_it - 1) // 2, body, 0)
        process(n_it - 1, 0)
        plsc.subcore_barrier()
        pltpu.sync_copy(acc.at[pl.ds(row0, ROWS_PER_TILE)],
                        s_hbm.at[cid, pl.ds(row0, ROWS_PER_TILE)])

    return scatter


_sc_gather = _make_gather(EH, 40)
_sc_scatter = _make_scatter(EH, 40)


N_IT_C = (E // NW) // CH_C


@functools.partial(
    pl.kernel,
    out_type=jax.ShapeDtypeStruct((NC, N_PAD, D), jnp.float32),
    mesh=_mesh,
    scratch_types=[
        pltpu.VMEM((N_IT_C, CH_C), jnp.int32),
        pltpu.VMEM((CH_C, D), jnp.float32),
        pltpu.VMEM_SHARED((N_PAD, D), jnp.float32),
    ],
)
def _sc_counts(dst2_hbm, ones_hbm, zeros_hbm, c_hbm, idx_d2, bufo, acc):
    cid = lax.axis_index("c")
    sid = lax.axis_index("s")
    wid = cid * NS + sid
    row0 = sid * ROWS_PER_TILE
    pltpu.sync_copy(dst2_hbm.at[wid], idx_d2)
    pltpu.sync_copy(zeros_hbm, acc.at[pl.ds(row0, ROWS_PER_TILE)])
    pltpu.sync_copy(ones_hbm, bufo)
    plsc.subcore_barrier()

    def body(j, carry):
        pltpu.sync_copy(bufo, acc.at[idx_d2.at[j]], add=True)
        return carry

    lax.fori_loop(0, N_IT_C, body, 0)
    plsc.subcore_barrier()
    pltpu.sync_copy(acc.at[pl.ds(row0, ROWS_PER_TILE)],
                    c_hbm.at[cid, pl.ds(row0, ROWS_PER_TILE)])


BN = 1000
BE = 1280

_full = lambda shape: pl.BlockSpec(shape, lambda i: (0,) * len(shape))
_rows = lambda b, w: pl.BlockSpec((b, w), lambda i: (i, 0))


def _tc_node_tables(h, wa, ba, wb):
    def body(h_ref, wa_ref, ba_ref, wb_ref, a_ref, b_ref):
        hv = h_ref[...]
        a_ref[...] = jnp.dot(hv, wa_ref[...], preferred_element_type=jnp.float32) + ba_ref[...]
        b_ref[...] = jnp.dot(hv, wb_ref[...], preferred_element_type=jnp.float32)
    return pl.pallas_call(
        body,
        grid=(N // BN,),
        in_specs=[_rows(BN, D), _full((D, D)), _full((1, D)), _full((D, D))],
        out_specs=[_rows(BN, D), _rows(BN, D)],
        out_shape=(jax.ShapeDtypeStruct((N, D), jnp.float32),
                   jax.ShapeDtypeStruct((N, D), jnp.float32)),
    )(h, wa, ba, wb)


def _tc_edge_mlp(g, w_row, v_row, w2, b2):
    def body(g_ref, w_ref, v_ref, w2_ref, b2_ref, m_ref):
        wv = lax.dot_general(w_ref[0], v_ref[...],
                             (((0,), (0,)), ((), ())),
                             preferred_element_type=jnp.float32)
        m1 = jnp.maximum(g_ref[...] + wv, 0.0).astype(jnp.bfloat16)
        m2 = jnp.dot(m1, w2_ref[...], preferred_element_type=jnp.float32) + b2_ref[...]
        m_ref[...] = jnp.maximum(m2, 0.0)
    return pl.pallas_call(
        body,
        grid=(EH // BE,),
        in_specs=[_rows(BE, D), pl.BlockSpec((1, 1, BE), lambda i: (i, 0, 0)),
                  _full((1, D)), _full((D, D)), _full((1, D))],
        out_specs=_rows(BE, D),
        out_shape=jax.ShapeDtypeStruct((EH, D), jnp.float32),
    )(g, w_row, v_row, w2, b2)


def _node_core(h_ref, sparts, c0, c1, wn1a, wn1b, bn1, wn2, bn2):
    cnt = jnp.maximum(c0[...][:, :1] + c1[...][:, :1], 1.0)
    ssum = sparts[0][...] + sparts[1][...] + sparts[2][...] + sparts[3][...]
    aggr = ssum * (1.0 / cnt)
    hv = h_ref[...]
    u = jnp.dot(hv, wn1a[...], preferred_element_type=jnp.float32)
    u = u + jnp.dot(aggr, wn1b[...], preferred_element_type=jnp.float32) + bn1[...]
    u = jnp.maximum(u, 0.0)
    hn = jnp.dot(u, wn2[...], preferred_element_type=jnp.float32) + bn2[...]
    return jnp.maximum(hn, 0.0)


def _tc_node_update(h, s00, s01, s10, s11, c0, c1,
                    wn1a, wn1b, bn1, wn2, bn2, wa, ba, wb):
    def body(h_ref, s00, s01, s10, s11, c0, c1, wn1a, wn1b, bn1, wn2, bn2,
             wa_ref, ba_ref, wb_ref, h_out, a_out, b_out):
        hn = _node_core(h_ref, (s00, s01, s10, s11), c0, c1,
                        wn1a, wn1b, bn1, wn2, bn2)
        h_out[...] = hn
        a_out[...] = jnp.dot(hn, wa_ref[...], preferred_element_type=jnp.float32) + ba_ref[...]
        b_out[...] = jnp.dot(hn, wb_ref[...], preferred_element_type=jnp.float32)
    return pl.pallas_call(
        body,
        grid=(N // BN,),
        in_specs=[_rows(BN, D), _rows(BN, D), _rows(BN, D), _rows(BN, D),
                  _rows(BN, D), _rows(BN, D), _rows(BN, D),
                  _full((D, D)), _full((D, D)), _full((1, D)),
                  _full((D, D)), _full((1, D)),
                  _full((D, D)), _full((1, D)), _full((D, D))],
        out_specs=[_rows(BN, D), _rows(BN, D), _rows(BN, D)],
        out_shape=(jax.ShapeDtypeStruct((N, D), jnp.float32),
                   jax.ShapeDtypeStruct((N, D), jnp.float32),
                   jax.ShapeDtypeStruct((N, D), jnp.float32)),
    )(h, s00, s01, s10, s11, c0, c1, wn1a, wn1b, bn1, wn2, bn2, wa, ba, wb)


def _tc_node_final(h, s00, s01, s10, s11, c0, c1,
                   wn1a, wn1b, bn1, wn2, bn2, g, bta):
    def body(h_ref, s00, s01, s10, s11, c0, c1, wn1a, wn1b, bn1, wn2, bn2,
             g_ref, bta_ref, y_out):
        hn = _node_core(h_ref, (s00, s01, s10, s11), c0, c1,
                        wn1a, wn1b, bn1, wn2, bn2)
        mu = jnp.mean(hn, axis=1, keepdims=True)
        dlt = hn - mu
        var = jnp.mean(dlt * dlt, axis=1, keepdims=True)
        y_out[...] = dlt * lax.rsqrt(var + 1e-5) * g_ref[...] + bta_ref[...]
    return pl.pallas_call(
        body,
        grid=(N // BN,),
        in_specs=[_rows(BN, D), _rows(BN, D), _rows(BN, D), _rows(BN, D),
                  _rows(BN, D), _rows(BN, D), _rows(BN, D),
                  _full((D, D)), _full((D, D)), _full((1, D)),
                  _full((D, D)), _full((1, D)),
                  _full((1, D)), _full((1, D))],
        out_specs=_rows(BN, D),
        out_shape=jax.ShapeDtypeStruct((N, D), jnp.float32),
    )(h, s00, s01, s10, s11, c0, c1, wn1a, wn1b, bn1, wn2, bn2, g, bta)


def kernel(x, edge_index, edge_weight, params):
    src = edge_index[0].astype(jnp.int32)
    dst = edge_index[1].astype(jnp.int32)
    src_g = (src[:EH].reshape(NW, EH // (NW * 40), 40),
             src[EH:].reshape(NW, EH // (NW * 40), 40))
    dst_g = (dst[:EH].reshape(NW, EH // (NW * 40), 40),
             dst[EH:].reshape(NW, EH // (NW * 40), 40))
    dst_h = (dst[:EH].reshape(NW, EH // (NW * 40), 40),
             dst[EH:].reshape(NW, EH // (NW * 40), 40))
    w_h = (edge_weight[:EH].reshape(EH // BE, 1, BE),
           edge_weight[EH:].reshape(EH // BE, 1, BE))

    zeros_d = jnp.zeros((ROWS_PER_TILE, D), jnp.float32)
    ones_d = jnp.ones((CH_C, D), jnp.float32)

    dst_c = dst.reshape(NW, E // (NW * CH_C), CH_C)
    cpart = _sc_counts(dst_c, ones_d, zeros_d)
    c0, c1 = cpart[0, :N], cpart[1, :N]
    tok = cpart[0, :8, 0]

    layers = params["layers"]

    h = x
    we1 = layers[0]["We1"]
    a, b = _tc_node_tables(h, we1[:D], layers[0]["be1"].reshape(1, D), we1[D:2 * D])
    for i, p in enumerate(layers):
        v_row = p["We1"][2 * D:2 * D + 1]
        w2_bf = p["We2"].astype(jnp.bfloat16)
        b2 = p["be2"].reshape(1, D)

        g0 = _sc_gather(a, b, dst_g[0], src_g[0], tok)
        g1 = _sc_gather(a, b, dst_g[1], src_g[1], g0[:8, 0])
        m0 = _tc_edge_mlp(g0, w_h[0], v_row, w2_bf, b2)
        m1 = _tc_edge_mlp(g1, w_h[1], v_row, w2_bf, b2)
        sp0 = _sc_scatter(m0, dst_h[0], zeros_d, g1[:8, 0])
        sp1 = _sc_scatter(m1, dst_h[1], zeros_d, sp0[0, :8, 0])
        tok = sp1[0, :8, 0]

        wn1 = p["Wn1"]
        args = (h, sp0[0, :N], sp0[1, :N], sp1[0, :N], sp1[1, :N], c0, c1,
                wn1[:D], wn1[D:], p["bn1"].reshape(1, D),
                p["Wn2"], p["bn2"].reshape(1, D))
        if i + 1 < len(layers):
            nxt = layers[i + 1]
            we1n = nxt["We1"]
            h, a, b = _tc_node_update(*args, we1n[:D],
                                      nxt["be1"].reshape(1, D), we1n[D:2 * D])
        else:
            h = _tc_node_final(*args, params["ln_scale"].reshape(1, D),
                               params["ln_bias"].reshape(1, D))
    return h
